# baseline (speedup 1.0000x reference)
import functools

import jax
import jax.numpy as jnp
from jax import lax
from jax.experimental import pallas as pl
from jax.experimental.pallas import tpu as pltpu
from jax.experimental.pallas import tpu_sc as plsc

_NUM_CORES = 2
_NUM_SUBCORES = 16
_NW = _NUM_CORES * _NUM_SUBCORES

_SEQ = 4096
_DIM = 1024
_B_PER_W = _SEQ // _NW
_CHUNK = 16
_SIZES = [16, 16, 16, 16, 16, 16, 16, 16]
_NCHUNK = len(_SIZES)
_OFFS = [sum(_SIZES[:i]) for i in range(_NCHUNK)]
_NBUF = 6


def _build_gather():
    mesh = plsc.VectorSubcoreMesh(core_axis_name="c", subcore_axis_name="s")

    nbuf = min(_NBUF, _NCHUNK)

    @functools.partial(
        pl.kernel,
        mesh=mesh,
        out_type=jax.ShapeDtypeStruct((_SEQ, _DIM), jnp.float32),
        scratch_types=(
            [pltpu.VMEM((_B_PER_W,), jnp.int32)]
            + [pltpu.VMEM((_CHUNK, _DIM), jnp.float32)] * nbuf
            + [pltpu.SemaphoreType.DMA] * 2
        ),
    )
    def gather_kernel(table_hbm, idx_hbm, out_hbm, idx_v, *scr):
        bufs = scr[:nbuf]
        gsem, ssem = scr[nbuf], scr[nbuf + 1]
        wid = lax.axis_index("s") * _NUM_CORES + lax.axis_index("c")
        base = wid * _B_PER_W
        pltpu.sync_copy(idx_hbm.at[pl.ds(base, _B_PER_W)], idx_v)

        gathers = [None] * nbuf
        stores = [None] * nbuf

        def start_gather(c):
            b = c % nbuf
            sz = _SIZES[c]
            gathers[b] = pltpu.async_copy(
                table_hbm.at[idx_v.at[pl.ds(_OFFS[c], sz)]],
                bufs[b] if sz == _CHUNK else bufs[b].at[pl.ds(0, sz)],
                gsem)

        def start_store(c):
            b = c % nbuf
            sz = _SIZES[c]
            gathers[b].wait()
            stores[b] = pltpu.async_copy(
                bufs[b] if sz == _CHUNK else bufs[b].at[pl.ds(0, sz)],
                out_hbm.at[pl.ds(base + _OFFS[c], sz)],
                ssem)

        for c in range(_NCHUNK):
            b = c % nbuf
            if stores[b] is not None:
                stores[b].wait()
            start_gather(c)
            oc = c - (nbuf - 1)
            if oc >= 0:
                start_store(oc)
        for oc in range(max(0, _NCHUNK - nbuf + 1), _NCHUNK):
            start_store(oc)
        for oc in range(max(0, _NCHUNK - nbuf), _NCHUNK):
            stores[oc % nbuf].wait()

    return gather_kernel


_gather = _build_gather()


def kernel(x_tc, times_t, embeddings_tc):
    del x_tc
    return _gather(embeddings_tc, times_t.astype(jnp.int32))


# speedup vs baseline: 1.0129x; 1.0129x over previous
import functools

import jax
import jax.numpy as jnp
from jax import lax
from jax.experimental import pallas as pl
from jax.experimental.pallas import tpu as pltpu
from jax.experimental.pallas import tpu_sc as plsc

_NUM_CORES = 2
_NUM_SUBCORES = 16
_NW = _NUM_CORES * _NUM_SUBCORES

_SEQ = 4096
_DIM = 1024
_B_PER_W = _SEQ // _NW
_CHUNK = 16
_SIZES = [16, 16, 16, 16, 16, 16, 16, 16]
_NCHUNK = len(_SIZES)
_OFFS = [sum(_SIZES[:i]) for i in range(_NCHUNK)]
_NBUF = 6


def _build_gather():
    mesh = plsc.VectorSubcoreMesh(core_axis_name="c", subcore_axis_name="s")

    nbuf = min(_NBUF, _NCHUNK)

    @functools.partial(
        pl.kernel,
        mesh=mesh,
        out_type=jax.ShapeDtypeStruct((_SEQ, _DIM), jnp.float32),
        scratch_types=(
            [pltpu.VMEM((_B_PER_W,), jnp.int32)]
            + [pltpu.VMEM((_CHUNK, _DIM), jnp.float32)] * nbuf
            + [pltpu.SemaphoreType.DMA] * (2 * nbuf)
        ),
    )
    def gather_kernel(table_hbm, idx_hbm, out_hbm, idx_v, *scr):
        bufs = scr[:nbuf]
        gsems = scr[nbuf:2 * nbuf]
        ssems = scr[2 * nbuf:]
        wid = lax.axis_index("s") * _NUM_CORES + lax.axis_index("c")
        base = wid * _B_PER_W
        pltpu.sync_copy(idx_hbm.at[pl.ds(base, _B_PER_W)], idx_v)

        gathers = [None] * nbuf
        stores = [None] * nbuf

        def start_gather(c):
            b = c % nbuf
            sz = _SIZES[c]
            gathers[b] = pltpu.async_copy(
                table_hbm.at[idx_v.at[pl.ds(_OFFS[c], sz)]],
                bufs[b] if sz == _CHUNK else bufs[b].at[pl.ds(0, sz)],
                gsems[b])

        def start_store(c):
            b = c % nbuf
            sz = _SIZES[c]
            gathers[b].wait()
            stores[b] = pltpu.async_copy(
                bufs[b] if sz == _CHUNK else bufs[b].at[pl.ds(0, sz)],
                out_hbm.at[pl.ds(base + _OFFS[c], sz)],
                ssems[b])

        for c in range(_NCHUNK):
            b = c % nbuf
            if stores[b] is not None:
                stores[b].wait()
            start_gather(c)
            oc = c - (nbuf - 1)
            if oc >= 0:
                start_store(oc)
        for oc in range(max(0, _NCHUNK - nbuf + 1), _NCHUNK):
            start_store(oc)
        for oc in range(max(0, _NCHUNK - nbuf), _NCHUNK):
            stores[oc % nbuf].wait()

    return gather_kernel


_gather = _build_gather()


def kernel(x_tc, times_t, embeddings_tc):
    del x_tc
    return _gather(embeddings_tc, times_t.astype(jnp.int32))
